# Initial kernel scaffold; baseline (speedup 1.0000x reference)
import jax
import jax.numpy as jnp
from jax.experimental import pallas as pl


def kernel(input):
    raise NotImplementedError("write your pallas kernel here")


# speedup vs baseline: 189.8362x; 189.8362x over previous
import math

import jax
import jax.numpy as jnp
import numpy as np
from jax.experimental import pallas as pl
from jax.experimental.pallas import tpu as pltpu

SAMPLE_RATE = 16000
MAX_ORDER = 15
RIR_LENGTH = 3968
WINDOW_LENGTH = 81
SOUND_SPEED = 343.0
PAD = WINDOW_LENGTH // 2
B = 32
NROWS = RIR_LENGTH // 128
W = 256
CHUNK = 1024


def _build_tables():
    ind = np.arange(-MAX_ORDER, MAX_ORDER + 1)
    X, Y, Z = np.meshgrid(ind, ind, ind, indexing='ij')
    xyz = np.stack([X.ravel(), Y.ravel(), Z.ravel()], axis=-1)
    xyz = xyz[np.abs(xyz).sum(axis=-1) <= MAX_ORDER]
    exp_lo = np.abs(np.floor(xyz / 2.0))
    exp_hi = np.abs(np.floor((xyz + 1) / 2.0))
    s_real = xyz.shape[0]
    s_pad = ((s_real + CHUNK - 1) // CHUNK) * CHUNK
    odd = (xyz % 2) == 1
    coef_room = np.where(odd, xyz + 1.0, xyz).astype(np.float32)
    sign_src = np.where(odd, -1.0, 1.0).astype(np.float32)
    T = np.zeros((16, s_pad), np.float32)
    T[0:3, :s_real] = coef_room.T
    T[3:6, :s_real] = sign_src.T
    T[6:12:2, :s_real] = exp_lo.T
    T[7:12:2, :s_real] = exp_hi.T
    T[12, :s_real] = 1.0
    if s_pad > s_real:
        T[0:6, s_real:] = T[0:6, 0:1]
    c = np.arange(W, dtype=np.float64)
    ccb = np.broadcast_to(np.cos(2.0 * np.pi * c / (2 * PAD))[:, None],
                          (W, 128)).astype(np.float32).copy()
    ssb = np.broadcast_to(np.sin(2.0 * np.pi * c / (2 * PAD))[:, None],
                          (W, 128)).astype(np.float32).copy()
    return T, ccb, ssb, s_pad


_T_NP, _CCB_NP, _SSB_NP, S_PAD = _build_tables()
NCH = S_PAD // CHUNK
_INV_PI = np.float32(1.0 / math.pi)
_PI = np.float32(math.pi)
_HANN_W = np.float32(2.0 * math.pi / (2 * PAD))


def _rir_kernel(scal_ref, t_ref, ccb_ref, ssb_ref, out_ref):
    b = pl.program_id(0)
    room = [scal_ref[b, 0], scal_ref[b, 1], scal_ref[b, 2]]
    mic = [scal_ref[b, 3], scal_ref[b, 4], scal_ref[b, 5]]
    src = [scal_ref[b, 6], scal_ref[b, 7], scal_ref[b, 8]]
    ltr = [scal_ref[b, 9 + k] for k in range(6)]

    cc = ccb_ref[:, 0:1]
    ss = ssb_ref[:, 0:1]
    ci = jax.lax.broadcasted_iota(jnp.int32, (W, CHUNK), 0)
    cf = ci.astype(jnp.float32)
    alt = (1 - 2 * (ci & 1)).astype(jnp.float32)

    acc = jnp.zeros((32, W), jnp.float32)
    for ch in range(NCH):
        sl = slice(ch * CHUNK, (ch + 1) * CHUNK)
        dist2 = None
        for d in range(3):
            img = room[d] * t_ref[d:d + 1, sl] + src[d] * t_ref[3 + d:4 + d, sl]
            diff = img - mic[d]
            dist2 = diff * diff if dist2 is None else dist2 + diff * diff
        dist = jnp.sqrt(dist2)
        delay = dist * np.float32(SAMPLE_RATE) / np.float32(SOUND_SPEED)
        delay_i = jnp.ceil(delay)
        frac = delay_i - delay
        q = jnp.floor(delay_i * np.float32(1.0 / 128.0))
        r = delay_i - q * 128.0
        lg = None
        for k in range(6):
            term = ltr[k] * t_ref[6 + k:7 + k, sl]
            lg = term if lg is None else lg + term
        att = jnp.exp(lg)
        mask = t_ref[12:13, sl]
        wgt = jnp.where(mask > 0, att / dist, 0.0)
        sp = jnp.sin(_PI * frac) * _INV_PI
        sgn = 1.0 - 2.0 * (r - 2.0 * jnp.floor(r * 0.5))
        phi = frac - r - np.float32(PAD)
        ang = phi * _HANN_W
        cphi = jnp.cos(ang)
        sphi = jnp.sin(ang)

        x = cf + phi
        inwin = jnp.abs(x) <= np.float32(PAD)
        hann = 0.5 + 0.5 * (cc * cphi - ss * sphi)
        num = alt * (sp * sgn)
        sinc = jnp.where(x == 0.0, 1.0, num / x)
        p = jnp.where(inwin, hann * sinc * wgt, 0.0)

        rows = jax.lax.broadcasted_iota(jnp.int32, (32, CHUNK), 0)
        oh = jnp.where(rows.astype(jnp.float32) == q, 1.0, 0.0)
        acc = acc + jax.lax.dot_general(
            oh, p, (((1,), (1,)), ((), ())),
            preferred_element_type=jnp.float32)

    lo = acc[0:NROWS, 0:128]
    hi = jnp.concatenate(
        [jnp.zeros((1, 128), jnp.float32), acc[0:NROWS - 1, 128:W]], axis=0)
    out_ref[0] = lo + hi


def kernel(input):
    input = input.astype(jnp.float32)
    room = input[:, 0:3]
    mic = input[:, 3:6] * room
    src = input[:, 6:9] * room
    a = jnp.concatenate(
        [jnp.repeat(input[:, 9:10], 4, axis=1), input[:, 10:11],
         input[:, 11:12]], axis=1)
    a = a * 0.84 + 0.01
    ltr = 0.5 * jnp.log(1.0 - a)
    scal = jnp.concatenate(
        [room, mic, src, ltr, jnp.zeros((B, 1), jnp.float32)], axis=1)

    t_tab = jnp.asarray(_T_NP)
    ccb = jnp.asarray(_CCB_NP)
    ssb = jnp.asarray(_SSB_NP)

    rir = pl.pallas_call(
        _rir_kernel,
        grid=(B,),
        in_specs=[
            pl.BlockSpec(memory_space=pltpu.SMEM),
            pl.BlockSpec((16, S_PAD), lambda b: (0, 0)),
            pl.BlockSpec((W, 128), lambda b: (0, 0)),
            pl.BlockSpec((W, 128), lambda b: (0, 0)),
        ],
        out_specs=pl.BlockSpec((1, NROWS, 128), lambda b: (b, 0, 0)),
        out_shape=jax.ShapeDtypeStruct((B, NROWS, 128), jnp.float32),
    )(scal, t_tab, ccb, ssb)
    rir = rir.reshape(B, RIR_LENGTH)

    dist_ms = jnp.linalg.norm(mic - src, axis=1)
    toa = WINDOW_LENGTH // 2 + SAMPLE_RATE * dist_ms / SOUND_SPEED
    return (rir, toa)
